# baseline (speedup 1.0000x reference)
import functools

import jax
import jax.numpy as jnp
from jax import lax
from jax.experimental import pallas as pl
from jax.experimental.pallas import tpu as pltpu
from jax.experimental.pallas import tpu_sc as plsc

VOCAB = 1000000
EMBED_DIM = 64
BATCH = 16384
HIST_LEN = 50

NC = 2
NS = 16
NW = NC * NS

NBB = BATCH // 128
BB_PER_W = NBB // NW
NPAIR = HIST_LEN * BB_PER_W
DEPTH = 4


def _gather_kernel(xt_hbm, table_hbm, p_hbm, idx_v, rows_v, tiles_v, *sems):
    wid = lax.axis_index("s") * NC + lax.axis_index("c")
    bb0 = wid * BB_PER_W
    gsems, ssems = sems[:DEPTH], sems[DEPTH:]

    pltpu.sync_copy(xt_hbm.at[:, pl.ds(bb0, BB_PER_W)], idx_v)

    iota16 = lax.iota(jnp.int32, 16)

    def issue_gather(i, d):
        t = i // BB_PER_W
        pltpu.async_copy(table_hbm.at[idx_v.at[t, i % BB_PER_W]],
                         rows_v.at[d], gsems[d])

    def transpose(d):
        rr = rows_v.at[d]
        tt = tiles_v.at[d]

        @pl.loop(0, 128, step=16)
        def _(b0):
            rows = b0 + iota16
            for dblk in range(8):
                for din in range(8):
                    col = jnp.full((16,), dblk * 8 + din, jnp.int32)
                    tt[dblk, din, pl.ds(b0, 16)] = plsc.load_gather(
                        rr, [rows, col])

    def issue_write(i, d):
        t = i // BB_PER_W
        pltpu.async_copy(tiles_v.at[d],
                         p_hbm.at[t, :, bb0 + (i % BB_PER_W)], ssems[d])

    def wait_gather(d):
        pltpu.make_async_copy(table_hbm.at[pl.ds(0, 128)], rows_v.at[d],
                              gsems[d]).wait()

    def wait_write(d):
        pltpu.make_async_copy(tiles_v.at[d], p_hbm.at[0, :, 0],
                              ssems[d]).wait()

    for d in range(DEPTH):
        issue_gather(d, d)
    for j in range(DEPTH):
        wait_gather(j)
        transpose(j)
        issue_write(j, j)
        issue_gather(j + DEPTH, j)

    @pl.loop(DEPTH, NPAIR - DEPTH, step=DEPTH)
    def _(i):
        for d in range(DEPTH):
            j = i + d
            wait_gather(d)
            wait_write(d)
            transpose(d)
            issue_write(j, d)
            issue_gather(j + DEPTH, d)

    for d in range(DEPTH):
        j = NPAIR - DEPTH + d
        wait_gather(d)
        wait_write(d)
        transpose(d)
        issue_write(j, d)

    for d in range(DEPTH):
        wait_write(d)


@jax.jit
def kernel(x, table):
    xt = x.astype(jnp.int32).T.reshape(HIST_LEN, NBB, 128)
    mesh = plsc.VectorSubcoreMesh(core_axis_name="c", subcore_axis_name="s")
    p = pl.kernel(
        _gather_kernel,
        out_type=jax.ShapeDtypeStruct((HIST_LEN, 8, NBB, 8, 128),
                                      jnp.float32),
        mesh=mesh,
        scratch_types=[
            pltpu.VMEM((HIST_LEN, BB_PER_W, 128), jnp.int32),
            pltpu.VMEM((DEPTH, 128, EMBED_DIM), jnp.float32),
            pltpu.VMEM((DEPTH, 8, 8, 128), jnp.float32),
        ] + [pltpu.SemaphoreType.DMA] * (2 * DEPTH),
        compiler_params=pltpu.CompilerParams(use_tc_tiling_on_sc=False,
                                             needs_layout_passes=False),
    )(xt, table)
    return p.transpose(2, 4, 0, 1, 3).reshape(BATCH, HIST_LEN, EMBED_DIM)


# speedup vs baseline: 1.8414x; 1.8414x over previous
import functools

import jax
import jax.numpy as jnp
from jax import lax
from jax.experimental import pallas as pl
from jax.experimental.pallas import tpu as pltpu
from jax.experimental.pallas import tpu_sc as plsc

VOCAB = 1000000
EMBED_DIM = 64
BATCH = 16384
HIST_LEN = 50

NC = 2
NS = 16
NW = NC * NS

NBB = BATCH // 128
BB_PER_W = NBB // NW
NPAIR = HIST_LEN * BB_PER_W
DEPTH = 4


def _gather_kernel(xt_hbm, table_hbm, p_hbm, idx_v, rows_v, tiles_v, *sems):
    wid = lax.axis_index("s") * NC + lax.axis_index("c")
    bb0 = wid * BB_PER_W
    gsems, ssems = sems[:DEPTH], sems[DEPTH:]

    pltpu.sync_copy(xt_hbm.at[:, pl.ds(bb0, BB_PER_W)], idx_v)

    iota16 = lax.iota(jnp.int32, 16)

    def issue_gather(i, d):
        t = i // BB_PER_W
        pltpu.async_copy(table_hbm.at[idx_v.at[t, i % BB_PER_W]],
                         rows_v.at[d], gsems[d])

    def transpose(d):
        rr = rows_v.at[d]
        tt = tiles_v.at[d]
        row_idx = [d0 + iota16 for d0 in range(0, EMBED_DIM, 16)]

        @pl.loop(0, 128, unroll=8)
        def _(b):
            col = jnp.full((16,), b, jnp.int32)
            for k in range(EMBED_DIM // 16):
                plsc.store_scatter(tt, [row_idx[k], col],
                                   rr[b, pl.ds(k * 16, 16)])

    def issue_write(i, d):
        t = i // BB_PER_W
        for dblk in range(8):
            pltpu.async_copy(
                tiles_v.at[d, pl.ds(8 * dblk, 8), pl.ds(0, 128)],
                p_hbm.at[t, dblk, bb0 + (i % BB_PER_W)], ssems[d])

    def wait_gather(d):
        pltpu.make_async_copy(table_hbm.at[pl.ds(0, 128)], rows_v.at[d],
                              gsems[d]).wait()

    def wait_write(d):
        for _ in range(8):
            pltpu.make_async_copy(
                tiles_v.at[d, pl.ds(0, 8), pl.ds(0, 128)],
                p_hbm.at[0, 0, 0], ssems[d]).wait()

    for d in range(DEPTH):
        issue_gather(d, d)
    for j in range(DEPTH):
        wait_gather(j)
        transpose(j)
        issue_write(j, j)
        issue_gather(j + DEPTH, j)

    @pl.loop(DEPTH, NPAIR - DEPTH, step=DEPTH)
    def _(i):
        for d in range(DEPTH):
            j = i + d
            wait_gather(d)
            wait_write(d)
            transpose(d)
            issue_write(j, d)
            issue_gather(j + DEPTH, d)

    for d in range(DEPTH):
        j = NPAIR - DEPTH + d
        wait_gather(d)
        wait_write(d)
        transpose(d)
        issue_write(j, d)

    for d in range(DEPTH):
        wait_write(d)


@jax.jit
def kernel(x, table):
    xt = x.astype(jnp.int32).T.reshape(HIST_LEN, NBB, 128)
    mesh = plsc.VectorSubcoreMesh(core_axis_name="c", subcore_axis_name="s")
    p = pl.kernel(
        _gather_kernel,
        out_type=jax.ShapeDtypeStruct((HIST_LEN, 8, NBB, 8, 128),
                                      jnp.float32),
        mesh=mesh,
        scratch_types=[
            pltpu.VMEM((HIST_LEN, BB_PER_W, 128), jnp.int32),
            pltpu.VMEM((DEPTH, 128, EMBED_DIM), jnp.float32),
            pltpu.VMEM((DEPTH, EMBED_DIM, 129), jnp.float32),
        ] + [pltpu.SemaphoreType.DMA] * (2 * DEPTH),
        compiler_params=pltpu.CompilerParams(use_tc_tiling_on_sc=False,
                                             needs_layout_passes=False),
    )(xt, table)
    return p.transpose(2, 4, 0, 1, 3).reshape(BATCH, HIST_LEN, EMBED_DIM)
